# baseline (speedup 1.0000x reference)
import functools

import jax
import jax.numpy as jnp
from jax import lax
from jax.experimental import pallas as pl
from jax.experimental.pallas import tpu as pltpu
from jax.experimental.pallas import tpu_sc as plsc

N = 10000
E = 320000
D_PAD = 16
NC, NS = 2, 16
NW = NC * NS
CH = 128
NCHUNK = 80
GK = 8
NGROUP = NCHUNK // GK
EPW = NCHUNK * CH
E_PAD = NW * EPW
ROWS_PER_TILE = 632
ZROWS = ROWS_PER_TILE // 8
N_PAD = NS * ROWS_PER_TILE

_acc_shape = jax.ShapeDtypeStruct((NC, N_PAD, D_PAD), jnp.float32)


def _zero_fill(buf, nrows):
    z = jnp.zeros((D_PAD,), jnp.float32)

    def body(i, c):
        buf[i] = z
        return c

    lax.fori_loop(0, nrows, body, 0)


def _sc_aggregate_body(g_hbm, src_hbm, dst_hbm, out_hbm,
                       src_v, dst_v, rows_v, zero_v, acc_sh, gtab_sh, sem, ssem):
    cid = lax.axis_index("c")
    sid = lax.axis_index("s")
    wid = sid * NC + cid

    pltpu.sync_copy(src_hbm.at[wid], src_v)
    pltpu.sync_copy(dst_hbm.at[wid], dst_v)

    base = sid * ROWS_PER_TILE
    row_sl = pl.ds(base, ROWS_PER_TILE)
    pltpu.sync_copy(g_hbm.at[row_sl], gtab_sh.at[row_sl])

    _zero_fill(zero_v, ZROWS)
    for k in range(8):
        pltpu.async_copy(zero_v, acc_sh.at[pl.ds(base + k * ZROWS, ZROWS)],
                         ssem.at[k % GK])
    for k in range(8):
        pltpu.make_async_copy(
            zero_v, acc_sh.at[pl.ds(base + k * ZROWS, ZROWS)],
            ssem.at[k % GK]).wait()
    plsc.subcore_barrier()

    def tab(b):
        return gtab_sh if b % 2 == 0 else g_hbm

    for b in range(GK):
        pltpu.async_copy(tab(b).at[src_v.at[b]], rows_v.at[b], sem.at[b])

    def body(g, c):
        for b in range(GK):
            j = g * GK + b
            pltpu.make_async_copy(
                tab(b).at[src_v.at[j]], rows_v.at[b], sem.at[b]).wait()
            pltpu.async_copy(
                rows_v.at[b], acc_sh.at[dst_v.at[j]], ssem.at[b], add=True)
        for b in range(GK):
            j = g * GK + b
            pltpu.make_async_copy(
                rows_v.at[b], acc_sh.at[dst_v.at[j]], ssem.at[b]).wait()
            pltpu.async_copy(tab(b).at[src_v.at[j + GK]], rows_v.at[b], sem.at[b])
        return c

    lax.fori_loop(0, NGROUP - 1, body, 0)
    for b in range(GK):
        j = (NGROUP - 1) * GK + b
        pltpu.make_async_copy(
            tab(b).at[src_v.at[j]], rows_v.at[b], sem.at[b]).wait()
        pltpu.async_copy(
            rows_v.at[b], acc_sh.at[dst_v.at[j]], ssem.at[b], add=True)
    for b in range(GK):
        j = (NGROUP - 1) * GK + b
        pltpu.make_async_copy(
            rows_v.at[b], acc_sh.at[dst_v.at[j]], ssem.at[b]).wait()
    plsc.subcore_barrier()

    pltpu.sync_copy(acc_sh.at[pl.ds(base, ROWS_PER_TILE)],
                    out_hbm.at[cid, pl.ds(base, ROWS_PER_TILE)])


def _sc_degree_body(dst_hbm, out_hbm, dst_v, ones_v, zero_v, acc_sh, zsem):
    cid = lax.axis_index("c")
    sid = lax.axis_index("s")
    wid = sid * NC + cid

    pltpu.sync_copy(dst_hbm.at[wid], dst_v)

    one = jnp.full((D_PAD,), 1.0, jnp.float32)

    def obody(i, c):
        ones_v[i] = one
        return c

    lax.fori_loop(0, CH, obody, 0)

    base = sid * ROWS_PER_TILE
    _zero_fill(zero_v, ZROWS)
    for k in range(8):
        pltpu.async_copy(zero_v, acc_sh.at[pl.ds(base + k * ZROWS, ZROWS)],
                         zsem.at[k])
    for k in range(8):
        pltpu.make_async_copy(
            zero_v, acc_sh.at[pl.ds(base + k * ZROWS, ZROWS)],
            zsem.at[k]).wait()
    plsc.subcore_barrier()

    def body(j, c):
        pltpu.sync_copy(ones_v, acc_sh.at[dst_v.at[j]], add=True)
        return c

    lax.fori_loop(0, NCHUNK, body, 0)
    plsc.subcore_barrier()

    pltpu.sync_copy(acc_sh.at[pl.ds(base, ROWS_PER_TILE)],
                    out_hbm.at[cid, pl.ds(base, ROWS_PER_TILE)])


@functools.cache
def _sc_kernels():
    mesh = plsc.VectorSubcoreMesh(core_axis_name="c", subcore_axis_name="s",
                                  num_cores=NC, num_subcores=NS)
    params = pltpu.CompilerParams(use_tc_tiling_on_sc=False)
    sc_aggregate = pl.kernel(
        _sc_aggregate_body,
        out_type=_acc_shape,
        mesh=mesh,
        compiler_params=params,
        scratch_types=[
            pltpu.VMEM((NCHUNK, CH), jnp.int32),
            pltpu.VMEM((NCHUNK, CH), jnp.int32),
            pltpu.VMEM((GK, CH, D_PAD), jnp.float32),
            pltpu.VMEM((ZROWS, D_PAD), jnp.float32),
            pltpu.VMEM_SHARED((N_PAD, D_PAD), jnp.float32),
            pltpu.VMEM_SHARED((N_PAD, D_PAD), jnp.float32),
            pltpu.SemaphoreType.DMA((GK,)),
            pltpu.SemaphoreType.DMA((GK,)),
        ],
    )
    sc_degree = pl.kernel(
        _sc_degree_body,
        out_type=_acc_shape,
        mesh=mesh,
        compiler_params=params,
        scratch_types=[
            pltpu.VMEM((NCHUNK, CH), jnp.int32),
            pltpu.VMEM((CH, D_PAD), jnp.float32),
            pltpu.VMEM((ZROWS, D_PAD), jnp.float32),
            pltpu.VMEM_SHARED((N_PAD, D_PAD), jnp.float32),
            pltpu.SemaphoreType.DMA((8,)),
        ],
    )
    return sc_aggregate, sc_degree


P = N_PAD // 8
PN = N // 8


EROWS = E // CH
EROWS_PAD = NW * NCHUNK


def _tc_edges_body(e_ref, o_ref):
    o_ref[:, EROWS_PAD - 64:] = jnp.full((2, 64, CH), N, jnp.int32)
    o_ref[:, :EROWS] = e_ref[...]


_tc_edges = pl.pallas_call(
    _tc_edges_body,
    out_shape=jax.ShapeDtypeStruct((2, EROWS_PAD, CH), jnp.int32),
)


def _tc_mm_body(x_ref, w_ref, h_ref):
    h_ref[:PN] = jnp.dot(x_ref[...], w_ref[...],
                         preferred_element_type=jnp.float32)
    h_ref[PN:] = jnp.zeros((P - PN, 128), jnp.float32)


_tc_mm = pl.pallas_call(
    _tc_mm_body,
    out_shape=jax.ShapeDtypeStruct((P, 128), jnp.float32),
)


def _tc_scale_body(h_ref, d_ref, g_ref, dinv_ref):
    deg = d_ref[0] + d_ref[1] + 1.0
    dinv = lax.rsqrt(deg)
    dinv_ref[...] = dinv
    g_ref[...] = h_ref[...] * dinv


_tc_scale = pl.pallas_call(
    _tc_scale_body,
    out_shape=[jax.ShapeDtypeStruct((P, 128), jnp.float32),
               jax.ShapeDtypeStruct((P, 128), jnp.float32)],
)


def _tc_mid_body(a_ref, g_ref, dinv_ref, b_ref, w_ref, o_ref):
    dinv = dinv_ref[...]
    u = dinv * (a_ref[0] + a_ref[1] + g_ref[...]) + b_ref[...]
    h = jnp.maximum(u, 0.0)
    o_ref[...] = jnp.dot(h, w_ref[...], preferred_element_type=jnp.float32) * dinv


_tc_mid = pl.pallas_call(
    _tc_mid_body,
    out_shape=jax.ShapeDtypeStruct((P, 128), jnp.float32),
)


def _tc_last_body(a_ref, g_ref, dinv_ref, b_ref, o_ref):
    u = dinv_ref[...] * (a_ref[0] + a_ref[1] + g_ref[...]) + b_ref[...]
    o_ref[...] = u[:PN]


_tc_last = pl.pallas_call(
    _tc_last_body,
    out_shape=jax.ShapeDtypeStruct((PN, 128), jnp.float32),
)


def kernel(x, edge_index, W1, b1, W2, b2, W3, b3):
    ep = _tc_edges(edge_index.reshape(2, EROWS, CH))
    src_p = ep[0].reshape(NW, NCHUNK, CH)
    dst_p = ep[1].reshape(NW, NCHUNK, CH)
    eye8 = jnp.eye(8, dtype=jnp.float32)
    W1blk = jnp.kron(eye8, jnp.pad(W1, ((0, 0), (0, D_PAD - W1.shape[1]))))
    W2blk = jnp.kron(eye8, jnp.pad(
        W2, ((0, D_PAD - W2.shape[0]), (0, D_PAD - W2.shape[1]))))
    W3blk = jnp.kron(eye8, jnp.pad(
        W3, ((0, D_PAD - W3.shape[0]), (0, D_PAD - W3.shape[1]))))
    b1pk = jnp.tile(jnp.pad(b1, (0, D_PAD - b1.shape[0])), 8).reshape(1, 128)
    b2pk = jnp.tile(jnp.pad(b2, (0, D_PAD - b2.shape[0])), 8).reshape(1, 128)
    b3pk = jnp.tile(jnp.pad(b3, (0, D_PAD - b3.shape[0])), 8).reshape(1, 128)

    sc_aggregate, sc_degree = _sc_kernels()
    h1 = _tc_mm(x.reshape(PN, 8 * 128), W1blk)
    d = sc_degree(dst_p)
    g1, dinv = _tc_scale(h1, d.reshape(NC, P, 128))
    a1 = sc_aggregate(g1.reshape(N_PAD, D_PAD), src_p, dst_p)
    g2 = _tc_mid(a1.reshape(NC, P, 128), g1, dinv, b1pk, W2blk)
    a2 = sc_aggregate(g2.reshape(N_PAD, D_PAD), src_p, dst_p)
    g3 = _tc_mid(a2.reshape(NC, P, 128), g2, dinv, b2pk, W3blk)
    a3 = sc_aggregate(g3.reshape(N_PAD, D_PAD), src_p, dst_p)
    out = _tc_last(a3.reshape(NC, P, 128), g3, dinv, b3pk)
    return out.reshape(N, D_PAD)


# speedup vs baseline: 1.1642x; 1.1642x over previous
import functools

import jax
import jax.numpy as jnp
from jax import lax
from jax.experimental import pallas as pl
from jax.experimental.pallas import tpu as pltpu
from jax.experimental.pallas import tpu_sc as plsc

N = 10000
E = 320000
D_PAD = 16
NC, NS = 2, 16
NW = NC * NS
CH = 128
NCHUNK = 80
GK = 8
NGROUP = NCHUNK // GK
EPW = NCHUNK * CH
E_PAD = NW * EPW
ROWS_PER_TILE = 632
ZROWS = ROWS_PER_TILE // 8
N_PAD = NS * ROWS_PER_TILE

_acc_shape = jax.ShapeDtypeStruct((NC, N_PAD, D_PAD), jnp.float32)


def _zero_fill(buf, nrows):
    z = jnp.zeros((D_PAD,), jnp.float32)

    def body(i, c):
        buf[i] = z
        return c

    lax.fori_loop(0, nrows, body, 0)


def _sc_aggregate_body(g_hbm, src_hbm, dst_hbm, out_hbm,
                       src_v, dst_v, rows_v, zero_v, acc_sh, gtab_sh, sem, ssem):
    cid = lax.axis_index("c")
    sid = lax.axis_index("s")
    wid = sid * NC + cid

    pltpu.sync_copy(src_hbm.at[wid], src_v)
    pltpu.sync_copy(dst_hbm.at[wid], dst_v)

    base = sid * ROWS_PER_TILE
    row_sl = pl.ds(base, ROWS_PER_TILE)
    pltpu.sync_copy(g_hbm.at[row_sl], gtab_sh.at[row_sl])

    _zero_fill(zero_v, ZROWS)
    for k in range(8):
        pltpu.async_copy(zero_v, acc_sh.at[pl.ds(base + k * ZROWS, ZROWS)],
                         ssem.at[k % GK])
    for k in range(8):
        pltpu.make_async_copy(
            zero_v, acc_sh.at[pl.ds(base + k * ZROWS, ZROWS)],
            ssem.at[k % GK]).wait()
    plsc.subcore_barrier()

    def tab(b):
        return gtab_sh

    for b in range(GK):
        pltpu.async_copy(tab(b).at[src_v.at[b]], rows_v.at[b], sem.at[b])

    def body(g, c):
        for b in range(GK):
            j = g * GK + b
            pltpu.make_async_copy(
                tab(b).at[src_v.at[j]], rows_v.at[b], sem.at[b]).wait()
            pltpu.async_copy(
                rows_v.at[b], acc_sh.at[dst_v.at[j]], ssem.at[b], add=True)
        for b in range(GK):
            j = g * GK + b
            pltpu.make_async_copy(
                rows_v.at[b], acc_sh.at[dst_v.at[j]], ssem.at[b]).wait()
            pltpu.async_copy(tab(b).at[src_v.at[j + GK]], rows_v.at[b], sem.at[b])
        return c

    lax.fori_loop(0, NGROUP - 1, body, 0)
    for b in range(GK):
        j = (NGROUP - 1) * GK + b
        pltpu.make_async_copy(
            tab(b).at[src_v.at[j]], rows_v.at[b], sem.at[b]).wait()
        pltpu.async_copy(
            rows_v.at[b], acc_sh.at[dst_v.at[j]], ssem.at[b], add=True)
    for b in range(GK):
        j = (NGROUP - 1) * GK + b
        pltpu.make_async_copy(
            rows_v.at[b], acc_sh.at[dst_v.at[j]], ssem.at[b]).wait()
    plsc.subcore_barrier()

    pltpu.sync_copy(acc_sh.at[pl.ds(base, ROWS_PER_TILE)],
                    out_hbm.at[cid, pl.ds(base, ROWS_PER_TILE)])


def _sc_degree_body(dst_hbm, out_hbm, dst_v, ones_v, zero_v, acc_sh, zsem):
    cid = lax.axis_index("c")
    sid = lax.axis_index("s")
    wid = sid * NC + cid

    pltpu.sync_copy(dst_hbm.at[wid], dst_v)

    one = jnp.full((D_PAD,), 1.0, jnp.float32)

    def obody(i, c):
        ones_v[i] = one
        return c

    lax.fori_loop(0, CH, obody, 0)

    base = sid * ROWS_PER_TILE
    _zero_fill(zero_v, ZROWS)
    for k in range(8):
        pltpu.async_copy(zero_v, acc_sh.at[pl.ds(base + k * ZROWS, ZROWS)],
                         zsem.at[k])
    for k in range(8):
        pltpu.make_async_copy(
            zero_v, acc_sh.at[pl.ds(base + k * ZROWS, ZROWS)],
            zsem.at[k]).wait()
    plsc.subcore_barrier()

    def body(j, c):
        pltpu.sync_copy(ones_v, acc_sh.at[dst_v.at[j]], add=True)
        return c

    lax.fori_loop(0, NCHUNK, body, 0)
    plsc.subcore_barrier()

    pltpu.sync_copy(acc_sh.at[pl.ds(base, ROWS_PER_TILE)],
                    out_hbm.at[cid, pl.ds(base, ROWS_PER_TILE)])


@functools.cache
def _sc_kernels():
    mesh = plsc.VectorSubcoreMesh(core_axis_name="c", subcore_axis_name="s",
                                  num_cores=NC, num_subcores=NS)
    params = pltpu.CompilerParams(use_tc_tiling_on_sc=False)
    sc_aggregate = pl.kernel(
        _sc_aggregate_body,
        out_type=_acc_shape,
        mesh=mesh,
        compiler_params=params,
        scratch_types=[
            pltpu.VMEM((NCHUNK, CH), jnp.int32),
            pltpu.VMEM((NCHUNK, CH), jnp.int32),
            pltpu.VMEM((GK, CH, D_PAD), jnp.float32),
            pltpu.VMEM((ZROWS, D_PAD), jnp.float32),
            pltpu.VMEM_SHARED((N_PAD, D_PAD), jnp.float32),
            pltpu.VMEM_SHARED((N_PAD, D_PAD), jnp.float32),
            pltpu.SemaphoreType.DMA((GK,)),
            pltpu.SemaphoreType.DMA((GK,)),
        ],
    )
    sc_degree = pl.kernel(
        _sc_degree_body,
        out_type=_acc_shape,
        mesh=mesh,
        compiler_params=params,
        scratch_types=[
            pltpu.VMEM((NCHUNK, CH), jnp.int32),
            pltpu.VMEM((CH, D_PAD), jnp.float32),
            pltpu.VMEM((ZROWS, D_PAD), jnp.float32),
            pltpu.VMEM_SHARED((N_PAD, D_PAD), jnp.float32),
            pltpu.SemaphoreType.DMA((8,)),
        ],
    )
    return sc_aggregate, sc_degree


P = N_PAD // 8
PN = N // 8


EROWS = E // CH
EROWS_PAD = NW * NCHUNK


def _tc_edges_body(e_ref, o_ref):
    o_ref[:, EROWS_PAD - 64:] = jnp.full((2, 64, CH), N, jnp.int32)
    o_ref[:, :EROWS] = e_ref[...]


_tc_edges = pl.pallas_call(
    _tc_edges_body,
    out_shape=jax.ShapeDtypeStruct((2, EROWS_PAD, CH), jnp.int32),
)


def _tc_mm_body(x_ref, w_ref, h_ref):
    h_ref[:PN] = jnp.dot(x_ref[...], w_ref[...],
                         preferred_element_type=jnp.float32)
    h_ref[PN:] = jnp.zeros((P - PN, 128), jnp.float32)


_tc_mm = pl.pallas_call(
    _tc_mm_body,
    out_shape=jax.ShapeDtypeStruct((P, 128), jnp.float32),
)


def _tc_scale_body(h_ref, d_ref, g_ref, dinv_ref):
    deg = d_ref[0] + d_ref[1] + 1.0
    dinv = lax.rsqrt(deg)
    dinv_ref[...] = dinv
    g_ref[...] = h_ref[...] * dinv


_tc_scale = pl.pallas_call(
    _tc_scale_body,
    out_shape=[jax.ShapeDtypeStruct((P, 128), jnp.float32),
               jax.ShapeDtypeStruct((P, 128), jnp.float32)],
)


def _tc_mid_body(a_ref, g_ref, dinv_ref, b_ref, w_ref, o_ref):
    dinv = dinv_ref[...]
    u = dinv * (a_ref[0] + a_ref[1] + g_ref[...]) + b_ref[...]
    h = jnp.maximum(u, 0.0)
    o_ref[...] = jnp.dot(h, w_ref[...], preferred_element_type=jnp.float32) * dinv


_tc_mid = pl.pallas_call(
    _tc_mid_body,
    out_shape=jax.ShapeDtypeStruct((P, 128), jnp.float32),
)


def _tc_last_body(a_ref, g_ref, dinv_ref, b_ref, o_ref):
    u = dinv_ref[...] * (a_ref[0] + a_ref[1] + g_ref[...]) + b_ref[...]
    o_ref[...] = u[:PN]


_tc_last = pl.pallas_call(
    _tc_last_body,
    out_shape=jax.ShapeDtypeStruct((PN, 128), jnp.float32),
)


def kernel(x, edge_index, W1, b1, W2, b2, W3, b3):
    ep = _tc_edges(edge_index.reshape(2, EROWS, CH))
    src_p = ep[0].reshape(NW, NCHUNK, CH)
    dst_p = ep[1].reshape(NW, NCHUNK, CH)
    eye8 = jnp.eye(8, dtype=jnp.float32)
    W1blk = jnp.kron(eye8, jnp.pad(W1, ((0, 0), (0, D_PAD - W1.shape[1]))))
    W2blk = jnp.kron(eye8, jnp.pad(
        W2, ((0, D_PAD - W2.shape[0]), (0, D_PAD - W2.shape[1]))))
    W3blk = jnp.kron(eye8, jnp.pad(
        W3, ((0, D_PAD - W3.shape[0]), (0, D_PAD - W3.shape[1]))))
    b1pk = jnp.tile(jnp.pad(b1, (0, D_PAD - b1.shape[0])), 8).reshape(1, 128)
    b2pk = jnp.tile(jnp.pad(b2, (0, D_PAD - b2.shape[0])), 8).reshape(1, 128)
    b3pk = jnp.tile(jnp.pad(b3, (0, D_PAD - b3.shape[0])), 8).reshape(1, 128)

    sc_aggregate, sc_degree = _sc_kernels()
    h1 = _tc_mm(x.reshape(PN, 8 * 128), W1blk)
    d = sc_degree(dst_p)
    g1, dinv = _tc_scale(h1, d.reshape(NC, P, 128))
    a1 = sc_aggregate(g1.reshape(N_PAD, D_PAD), src_p, dst_p)
    g2 = _tc_mid(a1.reshape(NC, P, 128), g1, dinv, b1pk, W2blk)
    a2 = sc_aggregate(g2.reshape(N_PAD, D_PAD), src_p, dst_p)
    g3 = _tc_mid(a2.reshape(NC, P, 128), g2, dinv, b2pk, W3blk)
    a3 = sc_aggregate(g3.reshape(N_PAD, D_PAD), src_p, dst_p)
    out = _tc_last(a3.reshape(NC, P, 128), g3, dinv, b3pk)
    return out.reshape(N, D_PAD)
